# baseline (speedup 1.0000x reference)
import functools

import jax
import jax.numpy as jnp
from jax import lax
from jax.experimental import pallas as pl
from jax.experimental.pallas import tpu as pltpu
from jax.experimental.pallas import tpu_sc as plsc

N, R, D, A, E = 10000, 3, 128, 64, 320000

NC = 2
NS = 16
DH = D // NC
PER_T = E // NS
CK = 64
PER_T_PAD = 20480
NCHUNK = PER_T_PAD // CK
STAGE = 40
NSTG = NCHUNK // STAGE
NBUF = 4
GAHEAD = 3
XROWS = 10240
XPT = XROWS // NS
XTAIL = N - (NS - 1) * XPT
AROWS = 10112
APT = AROWS // NS


def _sc_body(xh, srcp, dstp, out, xs, agg, src_b, dst_b, *bufs_and_sems):
    rows = list(bufs_and_sems[:NBUF])
    sg = list(bufs_and_sems[NBUF:2 * NBUF])
    ss = list(bufs_and_sems[2 * NBUF:3 * NBUF])
    c = lax.axis_index("c")
    s = lax.axis_index("s")

    def _gather(cc, b):
        pltpu.async_copy(xs.at[src_b.at[cc]], rows[b], sg[b])

    def _gather_wait(cc, b):
        pltpu.make_async_copy(xs.at[src_b.at[cc]], rows[b], sg[b]).wait()

    def _scatter(cc, b):
        pltpu.async_copy(rows[b], agg.at[dst_b.at[cc]], ss[b], add=True)

    def _scatter_wait(cc, b):
        pltpu.make_async_copy(rows[b], agg.at[dst_b.at[cc]], ss[b]).wait()

    for r in range(R):
        xrow = (c * R + r) * N + s * XPT

        @pl.when(s < NS - 1)
        def _():
            pltpu.sync_copy(xh.at[pl.ds(xrow, XPT)],
                            xs.at[pl.ds(s * XPT, XPT)])

        @pl.when(s == NS - 1)
        def _():
            pltpu.sync_copy(xh.at[pl.ds(xrow, XTAIL)],
                            xs.at[pl.ds(s * XPT, XTAIL)])

        def _zrow(i, _):
            def _zcol(k, _):
                rows[0][i, pl.ds(k * 16, 16)] = jnp.zeros((16,), jnp.float32)
                return 0
            return lax.fori_loop(0, DH // 16, _zcol, 0)
        lax.fori_loop(0, CK, _zrow, 0)
        abase = s * APT
        for k in range(APT // CK):
            pltpu.sync_copy(rows[0], agg.at[pl.ds(abase + k * CK, CK)])
        rem = APT - (APT // CK) * CK
        if rem:
            pltpu.sync_copy(rows[0].at[pl.ds(0, rem)],
                            agg.at[pl.ds(abase + (APT // CK) * CK, rem)])
        plsc.subcore_barrier()

        rb = (r * NS + s) * NCHUNK
        for stg in range(NSTG):
            pltpu.sync_copy(srcp.at[pl.ds(rb + stg * STAGE, STAGE)], src_b)
            pltpu.sync_copy(dstp.at[pl.ds(rb + stg * STAGE, STAGE)], dst_b)

            for p in range(GAHEAD):
                _gather(p, p)

            def _grp(j, _):
                for b in range(NBUF):
                    cc = NBUF * j + b
                    bn = (b + GAHEAD) % NBUF

                    @pl.when(cc >= NBUF - GAHEAD)
                    def _():
                        _scatter_wait(cc - (NBUF - GAHEAD), bn)

                    @pl.when(cc + GAHEAD < STAGE)
                    def _():
                        _gather(cc + GAHEAD, bn)
                    _gather_wait(cc, b)
                    _scatter(cc, b)
                return 0
            lax.fori_loop(0, STAGE // NBUF, _grp, 0)
            for t in range(NBUF - GAHEAD):
                _scatter_wait(STAGE - (NBUF - GAHEAD) + t,
                              (STAGE - (NBUF - GAHEAD) + t) % NBUF)
        plsc.subcore_barrier()

        orow = (c * R + r) * AROWS + abase
        pltpu.sync_copy(agg.at[pl.ds(abase, APT)],
                        out.at[pl.ds(orow, APT)])
        plsc.subcore_barrier()


def _make_sc_call():
    mesh = plsc.VectorSubcoreMesh(core_axis_name="c", subcore_axis_name="s")
    return functools.partial(
        pl.kernel, mesh=mesh,
        out_type=jax.ShapeDtypeStruct((NC * R * AROWS, DH), jnp.float32),
        scratch_types=(
            [pltpu.VMEM_SHARED((XROWS, DH), jnp.float32)]
            + [pltpu.VMEM_SHARED((AROWS, DH), jnp.float32)]
            + [pltpu.VMEM((STAGE, CK), jnp.int32)] * 2
            + [pltpu.VMEM((CK, DH), jnp.float32)] * NBUF
            + [pltpu.SemaphoreType.DMA] * (2 * NBUF)
        ),
    )(_sc_body)


def _tc_body(xT_ref, p_ref, w1_ref, b1_ref, w2_ref, b2_ref, aw1_ref, aw2_ref,
             out_ref):
    hs = []
    lgs = []
    for r in range(R):
        agg_r = jnp.concatenate([p_ref[0, r], p_ref[1, r]], axis=1)
        pre = xT_ref[r] + agg_r
        h1 = jnp.maximum(
            jnp.dot(pre, w1_ref[...], preferred_element_type=jnp.float32)
            + b1_ref[...], 0.0)
        h = jnp.maximum(
            jnp.dot(h1, w2_ref[...], preferred_element_type=jnp.float32)
            + b2_ref[...], 0.0)
        t = jnp.tanh(jnp.dot(h, aw1_ref[r], preferred_element_type=jnp.float32))
        lg = jnp.dot(t, aw2_ref[r], preferred_element_type=jnp.float32)
        hs.append(h)
        lgs.append(lg)
    m = jnp.maximum(jnp.maximum(lgs[0], lgs[1]), lgs[2])
    es = [jnp.exp(lg - m) for lg in lgs]
    den = es[0] + es[1] + es[2]
    for i in range(R):
        s_i = es[i] / den
        o = s_i[:, 0:1] * hs[0]
        for j in range(1, R):
            o = o + s_i[:, j:j + 1] * hs[j]
        out_ref[:, i, :] = o


def _dense(xT, partials, gin_w1, gin_b1, gin_w2, gin_b2, att_w1, att_w2):
    B = 1000
    grid = (N // B,)
    return pl.pallas_call(
        _tc_body,
        grid=grid,
        in_specs=[
            pl.BlockSpec((R, B, D), lambda i: (0, i, 0)),
            pl.BlockSpec((NC, R, B, DH), lambda i: (0, 0, i, 0)),
            pl.BlockSpec((D, D), lambda i: (0, 0)),
            pl.BlockSpec((1, D), lambda i: (0, 0)),
            pl.BlockSpec((D, D), lambda i: (0, 0)),
            pl.BlockSpec((1, D), lambda i: (0, 0)),
            pl.BlockSpec((R, D, A), lambda i: (0, 0, 0)),
            pl.BlockSpec((R, A, R), lambda i: (0, 0, 0)),
        ],
        out_specs=pl.BlockSpec((B, R, D), lambda i: (i, 0, 0)),
        out_shape=jax.ShapeDtypeStruct((N, R, D), jnp.float32),
    )(xT, partials, gin_w1, gin_b1.reshape(1, D), gin_w2,
      gin_b2.reshape(1, D), att_w1, att_w2)


def kernel(node_feat, edge_index, gin_w1, gin_b1, gin_w2, gin_b2, att_w1,
           att_w2):
    xT = jnp.transpose(node_feat, (1, 0, 2))
    xh = jnp.stack([xT[:, :, :DH], xT[:, :, DH:]], axis=0)
    xh = xh.reshape(NC * R * N, DH)
    src = edge_index[:, 0, :].reshape(R, NS, PER_T)
    dst = edge_index[:, 1, :].reshape(R, NS, PER_T)
    pad = PER_T_PAD - PER_T
    srcp = jnp.pad(src, ((0, 0), (0, 0), (0, pad)))
    dstp = jnp.pad(dst, ((0, 0), (0, 0), (0, pad)),
                   constant_values=N)
    srcp = srcp.reshape(R * NS * NCHUNK, CK)
    dstp = dstp.reshape(R * NS * NCHUNK, CK)

    sc = _make_sc_call()
    part = sc(xh, srcp, dstp)
    part = part.reshape(NC, R, AROWS, DH)

    return _dense(xT, part, gin_w1, gin_b1, gin_w2, gin_b2, att_w1, att_w2)


# speedup vs baseline: 1.0991x; 1.0991x over previous
import functools

import jax
import jax.numpy as jnp
from jax import lax
from jax.experimental import pallas as pl
from jax.experimental.pallas import tpu as pltpu
from jax.experimental.pallas import tpu_sc as plsc

N, R, D, A, E = 10000, 3, 128, 64, 320000

NC = 2
NS = 16
DH = D // NC
PER_T = E // NS
CK = 64
PER_T_PAD = 20480
NCHUNK = PER_T_PAD // CK
STAGE = 40
NSTG = NCHUNK // STAGE
NBUF = 4
GAHEAD = 2
XROWS = 10240
XPT = XROWS // NS
XTAIL = N - (NS - 1) * XPT
AROWS = 10112
APT = AROWS // NS


def _sc_body(xh, srcp, dstp, out, xs, agg, src_b, dst_b, *bufs_and_sems):
    rows = list(bufs_and_sems[:NBUF])
    sg = list(bufs_and_sems[NBUF:2 * NBUF])
    ss = list(bufs_and_sems[2 * NBUF:3 * NBUF])
    c = lax.axis_index("c")
    s = lax.axis_index("s")

    def _gather(cc, b):
        pltpu.async_copy(xs.at[src_b.at[cc]], rows[b], sg[b])

    def _gather_wait(cc, b):
        pltpu.make_async_copy(xs.at[src_b.at[cc]], rows[b], sg[b]).wait()

    def _scatter(cc, b):
        pltpu.async_copy(rows[b], agg.at[dst_b.at[cc]], ss[b], add=True)

    def _scatter_wait(cc, b):
        pltpu.make_async_copy(rows[b], agg.at[dst_b.at[cc]], ss[b]).wait()

    for r in range(R):
        xrow = (c * R + r) * N + s * XPT

        @pl.when(s < NS - 1)
        def _():
            pltpu.sync_copy(xh.at[pl.ds(xrow, XPT)],
                            xs.at[pl.ds(s * XPT, XPT)])

        @pl.when(s == NS - 1)
        def _():
            pltpu.sync_copy(xh.at[pl.ds(xrow, XTAIL)],
                            xs.at[pl.ds(s * XPT, XTAIL)])

        def _zrow(i, _):
            def _zcol(k, _):
                rows[0][i, pl.ds(k * 16, 16)] = jnp.zeros((16,), jnp.float32)
                return 0
            return lax.fori_loop(0, DH // 16, _zcol, 0)
        lax.fori_loop(0, CK, _zrow, 0)
        abase = s * APT
        for k in range(APT // CK):
            pltpu.sync_copy(rows[0], agg.at[pl.ds(abase + k * CK, CK)])
        rem = APT - (APT // CK) * CK
        if rem:
            pltpu.sync_copy(rows[0].at[pl.ds(0, rem)],
                            agg.at[pl.ds(abase + (APT // CK) * CK, rem)])
        plsc.subcore_barrier()

        rb = (r * NS + s) * NCHUNK
        for stg in range(NSTG):
            pltpu.sync_copy(srcp.at[pl.ds(rb + stg * STAGE, STAGE)], src_b)
            pltpu.sync_copy(dstp.at[pl.ds(rb + stg * STAGE, STAGE)], dst_b)

            for p in range(GAHEAD):
                _gather(p, p)

            def _grp(j, _):
                for b in range(NBUF):
                    cc = NBUF * j + b
                    bn = (b + GAHEAD) % NBUF

                    @pl.when(cc >= NBUF - GAHEAD)
                    def _():
                        _scatter_wait(cc - (NBUF - GAHEAD), bn)

                    @pl.when(cc + GAHEAD < STAGE)
                    def _():
                        _gather(cc + GAHEAD, bn)
                    _gather_wait(cc, b)
                    _scatter(cc, b)
                return 0
            lax.fori_loop(0, STAGE // NBUF, _grp, 0)
            for t in range(NBUF - GAHEAD):
                _scatter_wait(STAGE - (NBUF - GAHEAD) + t,
                              (STAGE - (NBUF - GAHEAD) + t) % NBUF)
        plsc.subcore_barrier()

        orow = (c * R + r) * AROWS + abase
        pltpu.sync_copy(agg.at[pl.ds(abase, APT)],
                        out.at[pl.ds(orow, APT)])
        plsc.subcore_barrier()


def _make_sc_call():
    mesh = plsc.VectorSubcoreMesh(core_axis_name="c", subcore_axis_name="s")
    return functools.partial(
        pl.kernel, mesh=mesh,
        out_type=jax.ShapeDtypeStruct((NC * R * AROWS, DH), jnp.float32),
        scratch_types=(
            [pltpu.VMEM_SHARED((XROWS, DH), jnp.float32)]
            + [pltpu.VMEM_SHARED((AROWS, DH), jnp.float32)]
            + [pltpu.VMEM((STAGE, CK), jnp.int32)] * 2
            + [pltpu.VMEM((CK, DH), jnp.float32)] * NBUF
            + [pltpu.SemaphoreType.DMA] * (2 * NBUF)
        ),
    )(_sc_body)


def _tc_body(xT_ref, p_ref, w1_ref, b1_ref, w2_ref, b2_ref, aw1_ref, aw2_ref,
             out_ref):
    hs = []
    lgs = []
    for r in range(R):
        agg_r = jnp.concatenate([p_ref[0, r], p_ref[1, r]], axis=1)
        pre = xT_ref[r] + agg_r
        h1 = jnp.maximum(
            jnp.dot(pre, w1_ref[...], preferred_element_type=jnp.float32)
            + b1_ref[...], 0.0)
        h = jnp.maximum(
            jnp.dot(h1, w2_ref[...], preferred_element_type=jnp.float32)
            + b2_ref[...], 0.0)
        t = jnp.tanh(jnp.dot(h, aw1_ref[r], preferred_element_type=jnp.float32))
        lg = jnp.dot(t, aw2_ref[r], preferred_element_type=jnp.float32)
        hs.append(h)
        lgs.append(lg)
    m = jnp.maximum(jnp.maximum(lgs[0], lgs[1]), lgs[2])
    es = [jnp.exp(lg - m) for lg in lgs]
    den = es[0] + es[1] + es[2]
    for i in range(R):
        s_i = es[i] / den
        o = s_i[:, 0:1] * hs[0]
        for j in range(1, R):
            o = o + s_i[:, j:j + 1] * hs[j]
        out_ref[:, i, :] = o


def _dense(xT, partials, gin_w1, gin_b1, gin_w2, gin_b2, att_w1, att_w2):
    B = 1000
    grid = (N // B,)
    return pl.pallas_call(
        _tc_body,
        grid=grid,
        in_specs=[
            pl.BlockSpec((R, B, D), lambda i: (0, i, 0)),
            pl.BlockSpec((NC, R, B, DH), lambda i: (0, 0, i, 0)),
            pl.BlockSpec((D, D), lambda i: (0, 0)),
            pl.BlockSpec((1, D), lambda i: (0, 0)),
            pl.BlockSpec((D, D), lambda i: (0, 0)),
            pl.BlockSpec((1, D), lambda i: (0, 0)),
            pl.BlockSpec((R, D, A), lambda i: (0, 0, 0)),
            pl.BlockSpec((R, A, R), lambda i: (0, 0, 0)),
        ],
        out_specs=pl.BlockSpec((B, R, D), lambda i: (i, 0, 0)),
        out_shape=jax.ShapeDtypeStruct((N, R, D), jnp.float32),
    )(xT, partials, gin_w1, gin_b1.reshape(1, D), gin_w2,
      gin_b2.reshape(1, D), att_w1, att_w2)


def kernel(node_feat, edge_index, gin_w1, gin_b1, gin_w2, gin_b2, att_w1,
           att_w2):
    xT = jnp.transpose(node_feat, (1, 0, 2))
    xh = jnp.stack([xT[:, :, :DH], xT[:, :, DH:]], axis=0)
    xh = xh.reshape(NC * R * N, DH)
    src = edge_index[:, 0, :].reshape(R, NS, PER_T)
    dst = edge_index[:, 1, :].reshape(R, NS, PER_T)
    pad = PER_T_PAD - PER_T
    srcp = jnp.pad(src, ((0, 0), (0, 0), (0, pad)))
    dstp = jnp.pad(dst, ((0, 0), (0, 0), (0, pad)),
                   constant_values=N)
    srcp = srcp.reshape(R * NS * NCHUNK, CK)
    dstp = dstp.reshape(R * NS * NCHUNK, CK)

    sc = _make_sc_call()
    part = sc(xh, srcp, dstp)
    part = part.reshape(NC, R, AROWS, DH)

    return _dense(xT, part, gin_w1, gin_b1, gin_w2, gin_b2, att_w1, att_w2)
